# baseline (speedup 1.0000x reference)
import functools

import jax
import jax.numpy as jnp
from jax import lax
from jax.experimental import pallas as pl
from jax.experimental.pallas import tpu as pltpu
from jax.experimental.pallas import tpu_sc as plsc

_VOCAB = 100000
_DIM = 64
_CTX = 20
_BATCH = 1024
_NIDX = _CTX * _BATCH

_SC_CORES = 2
_SC_SUBCORES = 16
_SC_WORKERS = _SC_CORES * _SC_SUBCORES
_IDX_PER_WORKER = _NIDX // _SC_WORKERS

_V_TILE = 2048
_NV = -(-_VOCAB // _V_TILE)
_VOCAB_PAD = _NV * _V_TILE


_DIM_PAD = 128


def _sc_gather(emb_table_padded, idx_flat):
    mesh = plsc.VectorSubcoreMesh(core_axis_name="c", subcore_axis_name="s")

    @functools.partial(
        pl.kernel,
        mesh=mesh,
        out_type=jax.ShapeDtypeStruct((_NIDX, _DIM_PAD), emb_table_padded.dtype),
        scratch_types=[
            pltpu.VMEM((_IDX_PER_WORKER,), jnp.int32),
            pltpu.VMEM((_IDX_PER_WORKER, _DIM_PAD), emb_table_padded.dtype),
            pltpu.SemaphoreType.DMA,
        ],
    )
    def gather_kernel(table_hbm, idx_hbm, out_hbm, idx_v, rows_v, sem):
        wid = lax.axis_index("s") * _SC_CORES + lax.axis_index("c")
        base = wid * _IDX_PER_WORKER
        pltpu.sync_copy(idx_hbm.at[pl.ds(base, _IDX_PER_WORKER)], idx_v)
        pltpu.async_copy(table_hbm.at[idx_v], rows_v, sem).wait()
        pltpu.sync_copy(rows_v, out_hbm.at[pl.ds(base, _IDX_PER_WORKER)])

    return gather_kernel(emb_table_padded, idx_flat)


def _fused_body(gat_ref, w_ref, b_ref, out_ref, emb_ref, m_ref, l_ref):
    p = pl.program_id(0)
    v = pl.program_id(1)

    @pl.when((p == 0) & (v == 0))
    def _init():
        acc = gat_ref[pl.ds(0, _BATCH), :]
        for c in range(1, _CTX):
            acc = acc + gat_ref[pl.ds(c * _BATCH, _BATCH), :]
        emb_ref[...] = acc[:, :_DIM]
        m_ref[...] = jnp.full((_BATCH, 1), -jnp.inf, jnp.float32)
        l_ref[...] = jnp.zeros((_BATCH, 1), jnp.float32)

    logits = lax.dot_general(
        emb_ref[...], w_ref[...], (((1,), (1,)), ((), ())),
        preferred_element_type=jnp.float32,
    ) + b_ref[...]

    @pl.when(p == 0)
    def _accumulate():
        m_prev = m_ref[...]
        m_new = jnp.maximum(m_prev, jnp.max(logits, axis=1, keepdims=True))
        l_new = l_ref[...] * jnp.exp(m_prev - m_new) + jnp.sum(
            jnp.exp(logits - m_new), axis=1, keepdims=True)
        m_ref[...] = m_new
        l_ref[...] = l_new

    @pl.when(p == 1)
    def _emit():
        out_ref[...] = logits - (m_ref[...] + jnp.log(l_ref[...]))


def _fused_projection_logsoftmax(gathered, W, b2d):
    return pl.pallas_call(
        _fused_body,
        grid=(2, _NV),
        in_specs=[
            pl.BlockSpec((_NIDX, _DIM_PAD), lambda p, v: (0, 0)),
            pl.BlockSpec((_V_TILE, _DIM), lambda p, v: (v, 0)),
            pl.BlockSpec((1, _V_TILE), lambda p, v: (0, v)),
        ],
        out_specs=pl.BlockSpec((_BATCH, _V_TILE), lambda p, v: (0, v * p)),
        out_shape=jax.ShapeDtypeStruct((_BATCH, _VOCAB), jnp.float32),
        scratch_shapes=[
            pltpu.VMEM((_BATCH, _DIM), jnp.float32),
            pltpu.VMEM((_BATCH, 1), jnp.float32),
            pltpu.VMEM((_BATCH, 1), jnp.float32),
        ],
    )(gathered, W, b2d)


def kernel(inputs, emb_table, W, b):
    idx_flat = inputs.astype(jnp.int32).reshape(_NIDX)
    table_p = jnp.pad(emb_table, ((0, 0), (0, _DIM_PAD - _DIM)))
    gathered = _sc_gather(table_p, idx_flat)
    pad = _VOCAB_PAD - _VOCAB
    W_p = jnp.pad(W, ((0, pad), (0, 0)))
    b_p = jnp.pad(b.reshape(1, _VOCAB), ((0, 0), (0, pad)),
                  constant_values=-1e30)
    return _fused_projection_logsoftmax(gathered, W_p, b_p)
